# baseline (speedup 1.0000x reference)
import jax
import jax.numpy as jnp
from jax.experimental import pallas as pl
from jax.experimental.pallas import tpu as pltpu

_G = 64
_N = 1024
_K = 4096
_M = 4096
_TM = 128
_NUM_STEPS = 96
_HALF = _NUM_STEPS // 2


def _gmm_body(mt_ref, gid_ref, num_steps_ref, x_ref, w_ref, o_ref):
    del mt_ref, gid_ref
    t = pl.program_id(0) * _HALF + pl.program_id(1)

    @pl.when(t < num_steps_ref[0])
    def _():
        acc = jax.lax.dot_general(
            x_ref[...], w_ref[0],
            (((1,), (1,)), ((), ())),
            preferred_element_type=jnp.float32)
        o_ref[...] = acc.astype(jnp.bfloat16)


def _grouped_matmul(mt, gid, num_steps, lhs_sorted, rhs):
    grid_spec = pltpu.PrefetchScalarGridSpec(
        num_scalar_prefetch=3,
        grid=(2, _HALF),
        in_specs=[
            pl.BlockSpec((_TM, _K),
                         lambda c, i, mt, gid, ns: (mt[c * _HALF + i], 0)),
            pl.BlockSpec((1, _N, _K),
                         lambda c, i, mt, gid, ns: (gid[c * _HALF + i], 0, 0)),
        ],
        out_specs=pl.BlockSpec((_TM, _N),
                               lambda c, i, mt, gid, ns: (c * _HALF + i, 0)),
    )
    return pl.pallas_call(
        _gmm_body,
        out_shape=jax.ShapeDtypeStruct((_NUM_STEPS * _TM, _N), jnp.bfloat16),
        grid_spec=grid_spec,
        compiler_params=pltpu.CompilerParams(
            dimension_semantics=("parallel", "arbitrary")),
        name="grouped_matmul",
    )(mt, gid, num_steps, lhs_sorted, rhs)


def kernel(lhs, rhs, m_indices):
    m_indices = m_indices.astype(jnp.int32)

    sort_idx = jnp.argsort(m_indices).astype(jnp.int32)
    g_sorted = m_indices[sort_idx]
    g_ar = jnp.arange(_G, dtype=jnp.int32)
    row_start = jnp.searchsorted(g_sorted, g_ar, side='left').astype(jnp.int32)
    row_end = jnp.searchsorted(g_sorted, g_ar, side='right').astype(jnp.int32)
    counts = row_end - row_start

    nonempty = counts > 0
    first_tile = jnp.where(nonempty, row_start // _TM, 0)
    last_tile = jnp.where(nonempty, (row_end - 1) // _TM, -1)
    steps_pg = jnp.where(nonempty, last_tile - first_tile + 1, 0)
    step_cum = jnp.cumsum(steps_pg).astype(jnp.int32)
    step_start = (step_cum - steps_pg).astype(jnp.int32)
    num_steps = step_cum[_G - 1]

    t_ar = jnp.arange(_NUM_STEPS, dtype=jnp.int32)
    raw_g = jnp.clip(
        jnp.searchsorted(step_cum, t_ar, side='right'), 0, _G - 1
    ).astype(jnp.int32)
    last_g = raw_g[jnp.maximum(num_steps - 1, 0)]
    gid = jnp.where(t_ar < num_steps, raw_g, last_g).astype(jnp.int32)
    mt_raw = jnp.clip(first_tile[gid] + (t_ar - step_start[gid]),
                      0, _M // _TM - 1)
    mt_last = mt_raw[jnp.maximum(num_steps - 1, 0)]
    mt = jnp.where(t_ar < num_steps, mt_raw, mt_last).astype(jnp.int32)

    ranks = jnp.arange(_M, dtype=jnp.int32)
    g_of_rank = g_sorted
    tile_of_rank = ranks // _TM
    step_of_rank = step_start[g_of_rank] + (tile_of_rank
                                            - first_tile[g_of_rank])
    slot_sorted = step_of_rank * _TM + (ranks % _TM)
    slot_of_row = jnp.zeros((_M,), jnp.int32).at[sort_idx].set(slot_sorted)

    lhs_sorted = lhs.at[sort_idx].get(mode='promise_in_bounds',
                                      unique_indices=True)
    out_slots = _grouped_matmul(mt, gid, num_steps.reshape(1),
                                lhs_sorted, rhs)
    return out_slots.at[slot_of_row].get(mode='promise_in_bounds',
                                         unique_indices=True)
